# Initial kernel scaffold; baseline (speedup 1.0000x reference)
import jax
import jax.numpy as jnp
from jax.experimental import pallas as pl


def kernel(pix_to_face, bary_coords, faces, verts):
    raise NotImplementedError("write your pallas kernel here")


# speedup vs baseline: 4.2505x; 4.2505x over previous
import functools

import jax
import jax.numpy as jnp
from jax import lax
from jax.experimental import pallas as pl
from jax.experimental.pallas import tpu as pltpu
from jax.experimental.pallas import tpu_sc as plsc

NW = 32
_PARAMS = pltpu.CompilerParams(
    use_tc_tiling_on_sc=False, needs_layout_passes=False)
_MESH = dict(core_axis_name="c", subcore_axis_name="s")


def _wid():
    return lax.axis_index("s") * 2 + lax.axis_index("c")


def _i16(v):
    return jnp.full((16,), v, jnp.int32)


def _build_fv(faces_flat, verts8, Fp):
    mf = Fp // NW
    rows3 = 3 * mf
    hf = mf // 2
    mesh = plsc.VectorSubcoreMesh(**_MESH)

    @functools.partial(
        pl.kernel,
        mesh=mesh,
        out_type=jax.ShapeDtypeStruct((Fp, 16), jnp.float32),
        compiler_params=_PARAMS,
        scratch_types=[
            pltpu.VMEM((rows3,), jnp.int32),
            pltpu.VMEM((rows3, 8), jnp.float32),
            pltpu.VMEM((hf, 16), jnp.float32),
            pltpu.SemaphoreType.DMA,
        ],
    )
    def build(ff_hbm, v8_hbm, fv_hbm, idx_v, rows_v, fv_v, sem):
        base = _wid() * mf
        pltpu.sync_copy(ff_hbm.at[pl.ds(3 * base, rows3)], idx_v)
        pltpu.async_copy(v8_hbm.at[idx_v], rows_v, sem).wait()
        for h in range(2):
            def group(g, _):
                lanes = g * 16 + lax.iota(jnp.int32, 16)
                rbase = 3 * (h * hf + lanes)
                for j in range(3):
                    for c in range(3):
                        val = plsc.load_gather(rows_v, [rbase + _i16(j), _i16(c)])
                        plsc.store_scatter(fv_v, [lanes, _i16(3 * j + c)], val)
                    ids = plsc.load_gather(idx_v, [rbase + _i16(j)])
                    plsc.store_scatter(fv_v, [lanes, _i16(9 + j)],
                                       plsc.bitcast(ids, jnp.float32))
                return 0

            lax.fori_loop(0, hf // 16, group, 0)
            pltpu.sync_copy(fv_v, fv_hbm.at[pl.ds(base + h * hf, hf)])

    return build(faces_flat, verts8)


def _shade(pix, bary, fv, N, m):
    n_per = N // NW
    mesh = plsc.VectorSubcoreMesh(**_MESH)

    @functools.partial(
        pl.kernel,
        mesh=mesh,
        out_type=(
            jax.ShapeDtypeStruct((N, 4), jnp.float32),
            jax.ShapeDtypeStruct((N, 3), jnp.int32),
        ),
        compiler_params=_PARAMS,
        scratch_types=[
            pltpu.VMEM((m,), jnp.int32),
            pltpu.VMEM((m, 16), jnp.float32),
            pltpu.VMEM((m, 3), jnp.float32),
            pltpu.VMEM((m, 4), jnp.float32),
            pltpu.VMEM((m, 3), jnp.int32),
            pltpu.SemaphoreType.DMA,
        ],
    )
    def shade(pix_hbm, bary_hbm, fv_hbm, res_hbm, vfo_hbm,
              pix_v, fv_v, bary_v, res_v, vf_v, sem):
        wbase = _wid() * n_per

        def chunk(i, _):
            base = wbase + i * m
            pltpu.sync_copy(pix_hbm.at[pl.ds(base, m)], pix_v)
            cp_fv = pltpu.async_copy(fv_hbm.at[pix_v], fv_v, sem)
            pltpu.sync_copy(bary_hbm.at[pl.ds(base, m)], bary_v)
            cp_fv.wait()

            def group(g, _):
                s = g * 16
                rows = s + lax.iota(jnp.int32, 16)
                pv = pix_v[pl.ds(s, 16)]
                b = [plsc.load_gather(bary_v, [rows, _i16(j)]) for j in range(3)]
                for c in range(3):
                    acc = b[0] * plsc.load_gather(fv_v, [rows, _i16(c)])
                    for j in (1, 2):
                        acc = acc + b[j] * plsc.load_gather(
                            fv_v, [rows, _i16(3 * j + c)])
                    plsc.store_scatter(res_v, [rows, _i16(c)], acc)
                alpha = jnp.where(pv != -1, 1.0, 0.0).astype(jnp.float32)
                plsc.store_scatter(res_v, [rows, _i16(3)], alpha)
                for j in range(3):
                    ids = plsc.bitcast(
                        plsc.load_gather(fv_v, [rows, _i16(9 + j)]), jnp.int32)
                    plsc.store_scatter(vf_v, [rows, _i16(j)], ids)
                return 0

            lax.fori_loop(0, m // 16, group, 0)
            pltpu.sync_copy(res_v, res_hbm.at[pl.ds(base, m)])
            pltpu.sync_copy(vf_v, vfo_hbm.at[pl.ds(base, m)])
            return 0

        lax.fori_loop(0, n_per // m, chunk, 0)

    return shade(pix, bary, fv)


def kernel(pix_to_face, bary_coords, faces, verts):
    B, H, W, _ = pix_to_face.shape
    N = B * H * W
    Fn = faces.shape[0]

    pix = pix_to_face.reshape(N)
    bary = bary_coords.reshape(N, 3)

    Fp = -(-Fn // (NW * 32)) * (NW * 32)
    faces_flat = jnp.pad(faces, ((0, Fp - Fn), (0, 0))).reshape(3 * Fp)
    verts8 = jnp.pad(verts, ((0, 0), (0, 5)))

    fv = _build_fv(faces_flat, verts8, Fp)
    res, vf = _shade(pix, bary, fv, N, 2048)

    results = res.reshape(B, H, W, 4)
    vertex_faces = vf.reshape(B, H, W, 3)
    return (results, vertex_faces, bary_coords.reshape(B, H, W, 3))
